# Initial kernel scaffold; baseline (speedup 1.0000x reference)
import jax
import jax.numpy as jnp
from jax.experimental import pallas as pl


def kernel(x, edge_index, W_gcn, b_gcn, W_res, b_res, W_ih, W_hh, b_ih, b_hh, W_fc, b_fc):
    raise NotImplementedError("write your pallas kernel here")


# speedup vs baseline: 5.7322x; 5.7322x over previous
import functools

import jax
import jax.numpy as jnp
from jax import lax
from jax.experimental import pallas as pl
from jax.experimental.pallas import tpu as pltpu
from jax.experimental.pallas import tpu_sc as plsc

N_NODES = 10000
D_IN = 256
DH = 128
NACC = 10240
NACC2 = 5120
ZROWS = 160
NC = 2
NS = 16
SUB = 128
CHUNK_ROWS = 8


def _sc_mesh():
    return plsc.VectorSubcoreMesh(core_axis_name="c", subcore_axis_name="s")


def _deg_body(rows_per_tile, dst2, out, idx_d, ones_v, zbuf, acc, sem):
    c = lax.axis_index("c")
    s = lax.axis_index("s")
    nh = N_NODES // NC
    base_node = c * nh

    def _fill(i, _):
        for j in range(DH // 16):
            zbuf[i, pl.ds(j * 16, 16)] = jnp.zeros((16,), jnp.float32)
        return 0
    lax.fori_loop(0, ZROWS, _fill, 0)

    def _fill1(i, _):
        for j in range(DH // 16):
            ones_v[i, pl.ds(j * 16, 16)] = jnp.ones((16,), jnp.float32)
        return 0
    lax.fori_loop(0, SUB, _fill1, 0)

    for k in range(NACC2 // NS // ZROWS):
        pltpu.sync_copy(zbuf, acc.at[pl.ds(
            pl.multiple_of(s * (NACC2 // NS) + k * ZROWS, 8), ZROWS)])
    plsc.subcore_barrier()

    base = s * rows_per_tile

    def _chunk(k, _):
        r0 = pl.multiple_of(base + k * CHUNK_ROWS, CHUNK_ROWS)
        pltpu.sync_copy(dst2.at[pl.ds(r0, CHUNK_ROWS)], idx_d)
        for j in range(CHUNK_ROWS):
            for i in range(SUB // 16):
                v = idx_d[j, pl.ds(i * 16, 16)]
                loc = v - base_node
                oob = (loc < 0) | (loc >= nh)
                idx_d[j, pl.ds(i * 16, 16)] = jnp.where(oob, nh, loc)
        for j in range(CHUNK_ROWS):
            pltpu.sync_copy(ones_v, acc.at[idx_d.at[j]], add=True)
        return 0
    lax.fori_loop(0, rows_per_tile // CHUNK_ROWS, _chunk, 0)
    plsc.subcore_barrier()

    cp = NACC2 // NS
    last = nh - (NS - 1) * cp

    @pl.when(s < NS - 1)
    def _():
        pltpu.sync_copy(acc.at[pl.ds(pl.multiple_of(s * cp, 8), cp)],
                        out.at[pl.ds(pl.multiple_of(base_node + s * cp, 8), cp)])

    @pl.when(s == NS - 1)
    def _():
        pltpu.sync_copy(acc.at[pl.ds((NS - 1) * cp, last)],
                        out.at[pl.ds(pl.multiple_of(base_node + (NS - 1) * cp, 8), last)])


def _agg_body(rows_per_tile, src2, dst2, xs_h, out_h,
              idx_s, idx_d, rows_v, zbuf, acc, sem):
    c = lax.axis_index("c")
    s = lax.axis_index("s")
    nh = N_NODES // NC
    base_node = c * nh

    def _fill(i, _):
        for j in range(DH // 16):
            zbuf[i, pl.ds(j * 16, 16)] = jnp.zeros((16,), jnp.float32)
        return 0
    lax.fori_loop(0, ZROWS, _fill, 0)

    for k in range(NACC2 // NS // ZROWS):
        pltpu.sync_copy(zbuf, acc.at[pl.ds(
            pl.multiple_of(s * (NACC2 // NS) + k * ZROWS, 8), ZROWS)])
    plsc.subcore_barrier()

    base = s * rows_per_tile

    def _chunk(k, _):
        r0 = pl.multiple_of(base + k * CHUNK_ROWS, CHUNK_ROWS)
        pltpu.sync_copy(src2.at[pl.ds(r0, CHUNK_ROWS)], idx_s)
        pltpu.sync_copy(dst2.at[pl.ds(r0, CHUNK_ROWS)], idx_d)
        for j in range(CHUNK_ROWS):
            for i in range(SUB // 16):
                v = idx_d[j, pl.ds(i * 16, 16)]
                loc = v - base_node
                oob = (loc < 0) | (loc >= nh)
                idx_d[j, pl.ds(i * 16, 16)] = jnp.where(oob, nh, loc)
        for j in range(CHUNK_ROWS):
            pltpu.async_copy(xs_h.at[idx_s.at[j]], rows_v, sem).wait()
            pltpu.sync_copy(rows_v, acc.at[idx_d.at[j]], add=True)
        return 0
    lax.fori_loop(0, rows_per_tile // CHUNK_ROWS, _chunk, 0)
    plsc.subcore_barrier()

    cp = NACC2 // NS
    last = nh - (NS - 1) * cp

    @pl.when(s < NS - 1)
    def _():
        pltpu.sync_copy(acc.at[pl.ds(pl.multiple_of(s * cp, 8), cp)],
                        out_h.at[pl.ds(pl.multiple_of(base_node + s * cp, 8), cp)])

    @pl.when(s == NS - 1)
    def _():
        pltpu.sync_copy(acc.at[pl.ds((NS - 1) * cp, last)],
                        out_h.at[pl.ds(pl.multiple_of(base_node + (NS - 1) * cp, 8), last)])


def _prescale_body(x_ref, degp_ref, lo_ref, hi_ref):
    deg = 1.0 + degp_ref[:, 0:1]
    dinv = lax.rsqrt(deg)
    xs = x_ref[...] * dinv
    lo_ref[...] = xs[:, :DH]
    hi_ref[...] = xs[:, DH:]


def _dense_body(alo_ref, ahi_ref, xlo_ref, xhi_ref, degp_ref,
                wg_ref, bg_ref, wr_ref, br_ref, wih_ref,
                bih_ref, wfc_ref, bfc_ref, out_ref):
    dinv = lax.rsqrt(1.0 + degp_ref[:, 0:1])
    agg = jnp.concatenate(
        [alo_ref[...] + xlo_ref[...], ahi_ref[...] + xhi_ref[...]],
        axis=1) * dinv
    dn = (((1,), (1,)), ((), ()))
    t = jnp.dot(agg, wg_ref[...], preferred_element_type=jnp.float32) + bg_ref[...]
    spatial = jnp.maximum(t, 0.0)
    spatial = spatial + lax.dot_general(
        spatial, wr_ref[...], dn, preferred_element_type=jnp.float32) + br_ref[...]
    gates = lax.dot_general(
        spatial, wih_ref[...], dn, preferred_element_type=jnp.float32) + bih_ref[...]
    gi = gates[:, 0:512]
    gg = gates[:, 512:1024]
    go = gates[:, 1024:1536]
    cc = jax.nn.sigmoid(gi) * jnp.tanh(gg)
    h = jax.nn.sigmoid(go) * jnp.tanh(cc)
    out_ref[...] = lax.dot_general(
        h, wfc_ref[...], dn, preferred_element_type=jnp.float32) + bfc_ref[...]


def _prep_edges(edge_index, n):
    e = edge_index.shape[1]
    ep = -(-e // (NC * NS * SUB * CHUNK_ROWS)) * (NC * NS * SUB * CHUNK_ROWS)
    pad = ep - e
    src_p = jnp.concatenate([edge_index[0], jnp.zeros((pad,), jnp.int32)])
    dst_p = jnp.concatenate([edge_index[1], jnp.full((pad,), n, jnp.int32)])
    return (src_p.reshape(ep // SUB, SUB), dst_p.reshape(ep // SUB, SUB),
            ep // SUB)


def _deg_call(dst2, rows):
    deg_fn = pl.kernel(
        functools.partial(_deg_body, rows // NS),
        out_type=jax.ShapeDtypeStruct((N_NODES, DH), jnp.float32),
        mesh=_sc_mesh(),
        scratch_types=[
            pltpu.VMEM((CHUNK_ROWS, SUB), jnp.int32),
            pltpu.VMEM((SUB, DH), jnp.float32),
            pltpu.VMEM((ZROWS, DH), jnp.float32),
            pltpu.VMEM_SHARED((NACC2, DH), jnp.float32),
            pltpu.SemaphoreType.DMA,
        ],
    )
    return deg_fn(dst2)


def _agg_call(src2, dst2, xs_h, rows):
    agg_fn = pl.kernel(
        functools.partial(_agg_body, rows // NS),
        out_type=jax.ShapeDtypeStruct((N_NODES, DH), jnp.float32),
        mesh=_sc_mesh(),
        scratch_types=[
            pltpu.VMEM((CHUNK_ROWS, SUB), jnp.int32),
            pltpu.VMEM((CHUNK_ROWS, SUB), jnp.int32),
            pltpu.VMEM((SUB, DH), jnp.float32),
            pltpu.VMEM((ZROWS, DH), jnp.float32),
            pltpu.VMEM_SHARED((NACC2, DH), jnp.float32),
            pltpu.SemaphoreType.DMA,
        ],
    )
    return agg_fn(src2, dst2, xs_h)


def kernel(x, edge_index, W_gcn, b_gcn, W_res, b_res, W_ih, W_hh, b_ih, b_hh,
           W_fc, b_fc):
    n = x.shape[0]
    assert n == N_NODES

    src2, dst2, rows = _prep_edges(edge_index, n)

    degp = _deg_call(dst2, rows)

    blk = 1000
    grid = n // blk
    xs_lo, xs_hi = pl.pallas_call(
        _prescale_body,
        grid=(grid,),
        in_specs=[
            pl.BlockSpec((blk, D_IN), lambda i: (i, 0)),
            pl.BlockSpec((blk, DH), lambda i: (i, 0)),
        ],
        out_specs=[pl.BlockSpec((blk, DH), lambda i: (i, 0))] * 2,
        out_shape=[jax.ShapeDtypeStruct((n, DH), jnp.float32)] * 2,
    )(x, degp)

    aggs_lo = _agg_call(src2, dst2, xs_lo, rows)
    aggs_hi = _agg_call(src2, dst2, xs_hi, rows)

    w_ih3 = jnp.concatenate([W_ih[0:512], W_ih[1024:2048]], axis=0)
    b3 = (b_ih + b_hh)
    b_ih3 = jnp.concatenate([b3[0:512], b3[1024:2048]]).reshape(1, 1536)
    out = pl.pallas_call(
        _dense_body,
        grid=(grid,),
        in_specs=[pl.BlockSpec((blk, DH), lambda i: (i, 0))] * 4 + [
            pl.BlockSpec((blk, DH), lambda i: (i, 0)),
            pl.BlockSpec((D_IN, 512), lambda i: (0, 0)),
            pl.BlockSpec((1, 512), lambda i: (0, 0)),
            pl.BlockSpec((512, 512), lambda i: (0, 0)),
            pl.BlockSpec((1, 512), lambda i: (0, 0)),
            pl.BlockSpec((1536, 512), lambda i: (0, 0)),
            pl.BlockSpec((1, 1536), lambda i: (0, 0)),
            pl.BlockSpec((D_IN, 512), lambda i: (0, 0)),
            pl.BlockSpec((1, D_IN), lambda i: (0, 0)),
        ],
        out_specs=pl.BlockSpec((blk, D_IN), lambda i: (i, 0)),
        out_shape=jax.ShapeDtypeStruct((n, D_IN), jnp.float32),
    )(aggs_lo, aggs_hi, xs_lo, xs_hi, degp,
      W_gcn, b_gcn.reshape(1, 512), W_res, b_res.reshape(1, 512),
      w_ih3, b_ih3, W_fc, b_fc.reshape(1, D_IN))
    return out


# speedup vs baseline: 6.1052x; 1.0651x over previous
import functools

import jax
import jax.numpy as jnp
from jax import lax
from jax.experimental import pallas as pl
from jax.experimental.pallas import tpu as pltpu
from jax.experimental.pallas import tpu_sc as plsc

N_NODES = 10000
D_IN = 256
DH = 128
NACC = 10240
NACC2 = 5120
ZROWS = 160
NC = 2
NS = 16
SUB = 128
CHUNK_ROWS = 8


def _sc_mesh():
    return plsc.VectorSubcoreMesh(core_axis_name="c", subcore_axis_name="s")


def _deg_body(rows_per_tile, dst2, out, idx_d, ones_v, zbuf, acc, sem):
    c = lax.axis_index("c")
    s = lax.axis_index("s")
    nh = N_NODES // NC
    base_node = c * nh

    def _fill(i, _):
        for j in range(DH // 16):
            zbuf[i, pl.ds(j * 16, 16)] = jnp.zeros((16,), jnp.float32)
        return 0
    lax.fori_loop(0, ZROWS, _fill, 0)

    def _fill1(i, _):
        for j in range(DH // 16):
            ones_v[i, pl.ds(j * 16, 16)] = jnp.ones((16,), jnp.float32)
        return 0
    lax.fori_loop(0, SUB, _fill1, 0)

    for k in range(NACC2 // NS // ZROWS):
        pltpu.sync_copy(zbuf, acc.at[pl.ds(
            pl.multiple_of(s * (NACC2 // NS) + k * ZROWS, 8), ZROWS)])
    plsc.subcore_barrier()

    base = pl.multiple_of(s * rows_per_tile, 8)
    pltpu.sync_copy(dst2.at[pl.ds(base, rows_per_tile)], idx_d)

    def _remap(j, _):
        for i in range(SUB // 16):
            v = idx_d[j, pl.ds(i * 16, 16)]
            loc = v - base_node
            oob = (loc < 0) | (loc >= nh)
            idx_d[j, pl.ds(i * 16, 16)] = jnp.where(oob, nh, loc)
        return 0
    lax.fori_loop(0, rows_per_tile, _remap, 0)

    def _fire(j, _):
        pltpu.sync_copy(ones_v, acc.at[idx_d.at[j]], add=True)
        return 0
    lax.fori_loop(0, rows_per_tile, _fire, 0)
    plsc.subcore_barrier()

    cp = NACC2 // NS
    last = nh - (NS - 1) * cp

    @pl.when(s < NS - 1)
    def _():
        pltpu.sync_copy(acc.at[pl.ds(pl.multiple_of(s * cp, 8), cp)],
                        out.at[pl.ds(pl.multiple_of(base_node + s * cp, 8), cp)])

    @pl.when(s == NS - 1)
    def _():
        pltpu.sync_copy(acc.at[pl.ds((NS - 1) * cp, last)],
                        out.at[pl.ds(pl.multiple_of(base_node + (NS - 1) * cp, 8), last)])


def _agg_body(rows_per_tile, src2, dst2, xs_h, out_h,
              idx_s, idx_d, buf0, buf1, zbuf, acc, sem0, sem1):
    c = lax.axis_index("c")
    s = lax.axis_index("s")
    nh = N_NODES // NC
    base_node = c * nh

    def _fill(i, _):
        for j in range(DH // 16):
            zbuf[i, pl.ds(j * 16, 16)] = jnp.zeros((16,), jnp.float32)
        return 0
    lax.fori_loop(0, ZROWS, _fill, 0)

    for k in range(NACC2 // NS // ZROWS):
        pltpu.sync_copy(zbuf, acc.at[pl.ds(
            pl.multiple_of(s * (NACC2 // NS) + k * ZROWS, 8), ZROWS)])
    plsc.subcore_barrier()

    base = pl.multiple_of(s * rows_per_tile, 8)
    pltpu.sync_copy(src2.at[pl.ds(base, rows_per_tile)], idx_s)
    pltpu.async_copy(xs_h.at[idx_s.at[0]], buf0, sem0)
    pltpu.sync_copy(dst2.at[pl.ds(base, rows_per_tile)], idx_d)

    def _remap(j, _):
        for i in range(SUB // 16):
            v = idx_d[j, pl.ds(i * 16, 16)]
            loc = v - base_node
            oob = (loc < 0) | (loc >= nh)
            idx_d[j, pl.ds(i * 16, 16)] = jnp.where(oob, nh, loc)
        return 0
    lax.fori_loop(0, rows_per_tile, _remap, 0)

    def _unit(t, _):
        j0 = 2 * t
        pltpu.make_async_copy(xs_h.at[idx_s.at[j0]], buf0, sem0).wait()
        pltpu.async_copy(xs_h.at[idx_s.at[j0 + 1]], buf1, sem1)
        pltpu.sync_copy(buf0, acc.at[idx_d.at[j0]], add=True)
        pltpu.make_async_copy(xs_h.at[idx_s.at[j0 + 1]], buf1, sem1).wait()

        @pl.when(t < rows_per_tile // 2 - 1)
        def _():
            pltpu.async_copy(xs_h.at[idx_s.at[j0 + 2]], buf0, sem0)
        pltpu.sync_copy(buf1, acc.at[idx_d.at[j0 + 1]], add=True)
        return 0
    lax.fori_loop(0, rows_per_tile // 2, _unit, 0)
    plsc.subcore_barrier()

    cp = NACC2 // NS
    last = nh - (NS - 1) * cp

    @pl.when(s < NS - 1)
    def _():
        pltpu.sync_copy(acc.at[pl.ds(pl.multiple_of(s * cp, 8), cp)],
                        out_h.at[pl.ds(pl.multiple_of(base_node + s * cp, 8), cp)])

    @pl.when(s == NS - 1)
    def _():
        pltpu.sync_copy(acc.at[pl.ds((NS - 1) * cp, last)],
                        out_h.at[pl.ds(pl.multiple_of(base_node + (NS - 1) * cp, 8), last)])


def _prescale_body(x_ref, degp_ref, lo_ref, hi_ref):
    deg = 1.0 + degp_ref[:, 0:1]
    dinv = lax.rsqrt(deg)
    xs = x_ref[...] * dinv
    lo_ref[...] = xs[:, :DH]
    hi_ref[...] = xs[:, DH:]


def _dense_body(alo_ref, ahi_ref, xlo_ref, xhi_ref, degp_ref,
                wg_ref, bg_ref, wr_ref, br_ref, wih_ref,
                bih_ref, wfc_ref, bfc_ref, out_ref):
    dinv = lax.rsqrt(1.0 + degp_ref[:, 0:1])
    agg = jnp.concatenate(
        [alo_ref[...] + xlo_ref[...], ahi_ref[...] + xhi_ref[...]],
        axis=1) * dinv
    dn = (((1,), (1,)), ((), ()))
    t = jnp.dot(agg, wg_ref[...], preferred_element_type=jnp.float32) + bg_ref[...]
    spatial = jnp.maximum(t, 0.0)
    spatial = spatial + lax.dot_general(
        spatial, wr_ref[...], dn, preferred_element_type=jnp.float32) + br_ref[...]
    gates = lax.dot_general(
        spatial, wih_ref[...], dn, preferred_element_type=jnp.float32) + bih_ref[...]
    gi = gates[:, 0:512]
    gg = gates[:, 512:1024]
    go = gates[:, 1024:1536]
    cc = jax.nn.sigmoid(gi) * jnp.tanh(gg)
    h = jax.nn.sigmoid(go) * jnp.tanh(cc)
    out_ref[...] = lax.dot_general(
        h, wfc_ref[...], dn, preferred_element_type=jnp.float32) + bfc_ref[...]


def _prep_edges(edge_index, n):
    e = edge_index.shape[1]
    ep = -(-e // (NC * NS * SUB * CHUNK_ROWS)) * (NC * NS * SUB * CHUNK_ROWS)
    pad = ep - e
    src_p = jnp.concatenate([edge_index[0], jnp.zeros((pad,), jnp.int32)])
    dst_p = jnp.concatenate([edge_index[1], jnp.full((pad,), n, jnp.int32)])
    return (src_p.reshape(ep // SUB, SUB), dst_p.reshape(ep // SUB, SUB),
            ep // SUB)


def _deg_call(dst2, rows):
    deg_fn = pl.kernel(
        functools.partial(_deg_body, rows // NS),
        out_type=jax.ShapeDtypeStruct((N_NODES, DH), jnp.float32),
        mesh=_sc_mesh(),
        scratch_types=[
            pltpu.VMEM((rows // NS, SUB), jnp.int32),
            pltpu.VMEM((SUB, DH), jnp.float32),
            pltpu.VMEM((ZROWS, DH), jnp.float32),
            pltpu.VMEM_SHARED((NACC2, DH), jnp.float32),
            pltpu.SemaphoreType.DMA,
        ],
    )
    return deg_fn(dst2)


def _agg_call(src2, dst2, xs_h, rows):
    agg_fn = pl.kernel(
        functools.partial(_agg_body, rows // NS),
        out_type=jax.ShapeDtypeStruct((N_NODES, DH), jnp.float32),
        mesh=_sc_mesh(),
        scratch_types=[
            pltpu.VMEM((rows // NS, SUB), jnp.int32),
            pltpu.VMEM((rows // NS, SUB), jnp.int32),
            pltpu.VMEM((SUB, DH), jnp.float32),
            pltpu.VMEM((SUB, DH), jnp.float32),
            pltpu.VMEM((ZROWS, DH), jnp.float32),
            pltpu.VMEM_SHARED((NACC2, DH), jnp.float32),
            pltpu.SemaphoreType.DMA,
            pltpu.SemaphoreType.DMA,
        ],
    )
    return agg_fn(src2, dst2, xs_h)


def kernel(x, edge_index, W_gcn, b_gcn, W_res, b_res, W_ih, W_hh, b_ih, b_hh,
           W_fc, b_fc):
    n = x.shape[0]
    assert n == N_NODES

    src2, dst2, rows = _prep_edges(edge_index, n)

    degp = _deg_call(dst2, rows)

    blk = 1000
    grid = n // blk
    xs_lo, xs_hi = pl.pallas_call(
        _prescale_body,
        grid=(grid,),
        in_specs=[
            pl.BlockSpec((blk, D_IN), lambda i: (i, 0)),
            pl.BlockSpec((blk, DH), lambda i: (i, 0)),
        ],
        out_specs=[pl.BlockSpec((blk, DH), lambda i: (i, 0))] * 2,
        out_shape=[jax.ShapeDtypeStruct((n, DH), jnp.float32)] * 2,
    )(x, degp)

    aggs_lo = _agg_call(src2, dst2, xs_lo, rows)
    aggs_hi = _agg_call(src2, dst2, xs_hi, rows)

    w_ih3 = jnp.concatenate([W_ih[0:512], W_ih[1024:2048]], axis=0)
    b3 = (b_ih + b_hh)
    b_ih3 = jnp.concatenate([b3[0:512], b3[1024:2048]]).reshape(1, 1536)
    out = pl.pallas_call(
        _dense_body,
        grid=(grid,),
        in_specs=[pl.BlockSpec((blk, DH), lambda i: (i, 0))] * 4 + [
            pl.BlockSpec((blk, DH), lambda i: (i, 0)),
            pl.BlockSpec((D_IN, 512), lambda i: (0, 0)),
            pl.BlockSpec((1, 512), lambda i: (0, 0)),
            pl.BlockSpec((512, 512), lambda i: (0, 0)),
            pl.BlockSpec((1, 512), lambda i: (0, 0)),
            pl.BlockSpec((1536, 512), lambda i: (0, 0)),
            pl.BlockSpec((1, 1536), lambda i: (0, 0)),
            pl.BlockSpec((D_IN, 512), lambda i: (0, 0)),
            pl.BlockSpec((1, D_IN), lambda i: (0, 0)),
        ],
        out_specs=pl.BlockSpec((blk, D_IN), lambda i: (i, 0)),
        out_shape=jax.ShapeDtypeStruct((n, D_IN), jnp.float32),
    )(aggs_lo, aggs_hi, xs_lo, xs_hi, degp,
      W_gcn, b_gcn.reshape(1, 512), W_res, b_res.reshape(1, 512),
      w_ih3, b_ih3, W_fc, b_fc.reshape(1, D_IN))
    return out


# speedup vs baseline: 6.1800x; 1.0123x over previous
import functools

import jax
import jax.numpy as jnp
from jax import lax
from jax.experimental import pallas as pl
from jax.experimental.pallas import tpu as pltpu
from jax.experimental.pallas import tpu_sc as plsc

N_NODES = 10000
D_IN = 256
DH = 128
NACC = 10240
NACC2 = 5120
ZROWS = 160
NC = 2
NS = 16
SUB = 128
CHUNK_ROWS = 8
NBUF = 4


def _sc_mesh():
    return plsc.VectorSubcoreMesh(core_axis_name="c", subcore_axis_name="s")


def _deg_body(rows_per_tile, dst2, out, idx_d, ones_v, zbuf, acc, sem):
    c = lax.axis_index("c")
    s = lax.axis_index("s")
    nh = N_NODES // NC
    base_node = c * nh

    def _fill(i, _):
        for j in range(DH // 16):
            zbuf[i, pl.ds(j * 16, 16)] = jnp.zeros((16,), jnp.float32)
        return 0
    lax.fori_loop(0, ZROWS, _fill, 0)

    def _fill1(i, _):
        for j in range(DH // 16):
            ones_v[i, pl.ds(j * 16, 16)] = jnp.ones((16,), jnp.float32)
        return 0
    lax.fori_loop(0, SUB, _fill1, 0)

    for k in range(NACC2 // NS // ZROWS):
        pltpu.sync_copy(zbuf, acc.at[pl.ds(
            pl.multiple_of(s * (NACC2 // NS) + k * ZROWS, 8), ZROWS)])
    plsc.subcore_barrier()

    base = pl.multiple_of(s * rows_per_tile, 8)
    pltpu.sync_copy(dst2.at[pl.ds(base, rows_per_tile)], idx_d)

    def _remap(j, _):
        for i in range(SUB // 16):
            v = idx_d[j, pl.ds(i * 16, 16)]
            loc = v - base_node
            oob = (loc < 0) | (loc >= nh)
            idx_d[j, pl.ds(i * 16, 16)] = jnp.where(oob, nh, loc)
        return 0
    lax.fori_loop(0, rows_per_tile, _remap, 0)

    def _fire(j, _):
        pltpu.sync_copy(ones_v, acc.at[idx_d.at[j]], add=True)
        return 0
    lax.fori_loop(0, rows_per_tile, _fire, 0)
    plsc.subcore_barrier()

    cp = NACC2 // NS
    last = nh - (NS - 1) * cp

    @pl.when(s < NS - 1)
    def _():
        pltpu.sync_copy(acc.at[pl.ds(pl.multiple_of(s * cp, 8), cp)],
                        out.at[pl.ds(pl.multiple_of(base_node + s * cp, 8), cp)])

    @pl.when(s == NS - 1)
    def _():
        pltpu.sync_copy(acc.at[pl.ds((NS - 1) * cp, last)],
                        out.at[pl.ds(pl.multiple_of(base_node + (NS - 1) * cp, 8), last)])


def _agg_body(rows_per_tile, src2, dst2, xs_lo, xs_hi, out_lo, out_hi,
              idx_s, idx_d, buf0, buf1, buf2, buf3, acc,
              sem0, sem1, sem2, sem3):
    c = lax.axis_index("c")
    s = lax.axis_index("s")
    nh = N_NODES // NC
    base_node = c * nh
    bufs = (buf0, buf1, buf2, buf3)
    sems = (sem0, sem1, sem2, sem3)

    def _zero_acc():
        def _fill(i, _):
            for j in range(DH // 16):
                buf0[i, pl.ds(j * 16, 16)] = jnp.zeros((16,), jnp.float32)
            return 0
        lax.fori_loop(0, SUB, _fill, 0)
        b0 = pl.multiple_of(s * (NACC2 // NS), 8)
        pltpu.sync_copy(buf0, acc.at[pl.ds(b0, SUB)])
        pltpu.sync_copy(buf0, acc.at[pl.ds(b0 + SUB, SUB)])
        pltpu.sync_copy(buf0.at[pl.ds(0, NACC2 // NS - 2 * SUB)],
                        acc.at[pl.ds(b0 + 2 * SUB, NACC2 // NS - 2 * SUB)])

    _zero_acc()
    plsc.subcore_barrier()

    base = pl.multiple_of(s * rows_per_tile, 8)
    pltpu.sync_copy(src2.at[pl.ds(base, rows_per_tile)], idx_s)
    pltpu.sync_copy(dst2.at[pl.ds(base, rows_per_tile)], idx_d)

    def _remap(j, _):
        for i in range(SUB // 16):
            v = idx_d[j, pl.ds(i * 16, 16)]
            loc = v - base_node
            oob = (loc < 0) | (loc >= nh)
            idx_d[j, pl.ds(i * 16, 16)] = jnp.where(oob, nh, loc)
        return 0
    lax.fori_loop(0, rows_per_tile, _remap, 0)

    cp = NACC2 // NS
    last = nh - (NS - 1) * cp

    def _pass(xs_h, out_h):
        for b in range(NBUF):
            pltpu.async_copy(xs_h.at[idx_s.at[b]], bufs[b], sems[b])

        def _unit(t, _):
            for b in range(NBUF):
                j = NBUF * t + b
                pltpu.make_async_copy(xs_h.at[idx_s.at[j]], bufs[b],
                                      sems[b]).wait()
                pltpu.sync_copy(bufs[b], acc.at[idx_d.at[j]], add=True)

                @pl.when(j + NBUF < rows_per_tile)
                def _():
                    pltpu.async_copy(xs_h.at[idx_s.at[j + NBUF]], bufs[b],
                                     sems[b])
            return 0
        lax.fori_loop(0, rows_per_tile // NBUF, _unit, 0)
        plsc.subcore_barrier()

        @pl.when(s < NS - 1)
        def _():
            pltpu.sync_copy(acc.at[pl.ds(pl.multiple_of(s * cp, 8), cp)],
                            out_h.at[pl.ds(pl.multiple_of(base_node + s * cp, 8), cp)])

        @pl.when(s == NS - 1)
        def _():
            pltpu.sync_copy(acc.at[pl.ds((NS - 1) * cp, last)],
                            out_h.at[pl.ds(pl.multiple_of(base_node + (NS - 1) * cp, 8), last)])

    _pass(xs_lo, out_lo)
    _zero_acc()
    plsc.subcore_barrier()
    _pass(xs_hi, out_hi)


def _prescale_body(x_ref, degp_ref, lo_ref, hi_ref):
    deg = 1.0 + degp_ref[:, 0:1]
    dinv = lax.rsqrt(deg)
    xs = x_ref[...] * dinv
    lo_ref[...] = xs[:, :DH]
    hi_ref[...] = xs[:, DH:]


def _dense_body(alo_ref, ahi_ref, xlo_ref, xhi_ref, degp_ref,
                wg_ref, bg_ref, wr_ref, br_ref, wih_ref,
                bih_ref, wfc_ref, bfc_ref, out_ref):
    dinv = lax.rsqrt(1.0 + degp_ref[:, 0:1])
    agg = jnp.concatenate(
        [alo_ref[...] + xlo_ref[...], ahi_ref[...] + xhi_ref[...]],
        axis=1) * dinv
    dn = (((1,), (1,)), ((), ()))
    t = jnp.dot(agg, wg_ref[...], preferred_element_type=jnp.float32) + bg_ref[...]
    spatial = jnp.maximum(t, 0.0)
    spatial = spatial + lax.dot_general(
        spatial, wr_ref[...], dn, preferred_element_type=jnp.float32) + br_ref[...]
    gates = lax.dot_general(
        spatial, wih_ref[...], dn, preferred_element_type=jnp.float32) + bih_ref[...]
    gi = gates[:, 0:512]
    gg = gates[:, 512:1024]
    go = gates[:, 1024:1536]
    cc = jax.nn.sigmoid(gi) * jnp.tanh(gg)
    h = jax.nn.sigmoid(go) * jnp.tanh(cc)
    out_ref[...] = lax.dot_general(
        h, wfc_ref[...], dn, preferred_element_type=jnp.float32) + bfc_ref[...]


def _prep_edges(edge_index, n):
    e = edge_index.shape[1]
    ep = -(-e // (NC * NS * SUB * CHUNK_ROWS)) * (NC * NS * SUB * CHUNK_ROWS)
    pad = ep - e
    src_p = jnp.concatenate([edge_index[0], jnp.zeros((pad,), jnp.int32)])
    dst_p = jnp.concatenate([edge_index[1], jnp.full((pad,), n, jnp.int32)])
    return (src_p.reshape(ep // SUB, SUB), dst_p.reshape(ep // SUB, SUB),
            ep // SUB)


def _deg_call(dst2, rows):
    deg_fn = pl.kernel(
        functools.partial(_deg_body, rows // NS),
        out_type=jax.ShapeDtypeStruct((N_NODES, DH), jnp.float32),
        mesh=_sc_mesh(),
        scratch_types=[
            pltpu.VMEM((rows // NS, SUB), jnp.int32),
            pltpu.VMEM((SUB, DH), jnp.float32),
            pltpu.VMEM((ZROWS, DH), jnp.float32),
            pltpu.VMEM_SHARED((NACC2, DH), jnp.float32),
            pltpu.SemaphoreType.DMA,
        ],
    )
    return deg_fn(dst2)


def _agg_call(src2, dst2, xs_lo, xs_hi, rows):
    agg_fn = pl.kernel(
        functools.partial(_agg_body, rows // NS),
        out_type=[jax.ShapeDtypeStruct((N_NODES, DH), jnp.float32)] * 2,
        mesh=_sc_mesh(),
        scratch_types=[
            pltpu.VMEM((rows // NS, SUB), jnp.int32),
            pltpu.VMEM((rows // NS, SUB), jnp.int32),
            pltpu.VMEM((SUB, DH), jnp.float32),
            pltpu.VMEM((SUB, DH), jnp.float32),
            pltpu.VMEM((SUB, DH), jnp.float32),
            pltpu.VMEM((SUB, DH), jnp.float32),
            pltpu.VMEM_SHARED((NACC2, DH), jnp.float32),
            pltpu.SemaphoreType.DMA,
            pltpu.SemaphoreType.DMA,
            pltpu.SemaphoreType.DMA,
            pltpu.SemaphoreType.DMA,
        ],
    )
    return agg_fn(src2, dst2, xs_lo, xs_hi)


def kernel(x, edge_index, W_gcn, b_gcn, W_res, b_res, W_ih, W_hh, b_ih, b_hh,
           W_fc, b_fc):
    n = x.shape[0]
    assert n == N_NODES

    src2, dst2, rows = _prep_edges(edge_index, n)

    degp = _deg_call(dst2, rows)

    blk = 1000
    grid = n // blk
    xs_lo, xs_hi = pl.pallas_call(
        _prescale_body,
        grid=(grid,),
        in_specs=[
            pl.BlockSpec((blk, D_IN), lambda i: (i, 0)),
            pl.BlockSpec((blk, DH), lambda i: (i, 0)),
        ],
        out_specs=[pl.BlockSpec((blk, DH), lambda i: (i, 0))] * 2,
        out_shape=[jax.ShapeDtypeStruct((n, DH), jnp.float32)] * 2,
    )(x, degp)

    aggs_lo, aggs_hi = _agg_call(src2, dst2, xs_lo, xs_hi, rows)

    w_ih3 = jnp.concatenate([W_ih[0:512], W_ih[1024:2048]], axis=0)
    b3 = (b_ih + b_hh)
    b_ih3 = jnp.concatenate([b3[0:512], b3[1024:2048]]).reshape(1, 1536)
    out = pl.pallas_call(
        _dense_body,
        grid=(grid,),
        in_specs=[pl.BlockSpec((blk, DH), lambda i: (i, 0))] * 4 + [
            pl.BlockSpec((blk, DH), lambda i: (i, 0)),
            pl.BlockSpec((D_IN, 512), lambda i: (0, 0)),
            pl.BlockSpec((1, 512), lambda i: (0, 0)),
            pl.BlockSpec((512, 512), lambda i: (0, 0)),
            pl.BlockSpec((1, 512), lambda i: (0, 0)),
            pl.BlockSpec((1536, 512), lambda i: (0, 0)),
            pl.BlockSpec((1, 1536), lambda i: (0, 0)),
            pl.BlockSpec((D_IN, 512), lambda i: (0, 0)),
            pl.BlockSpec((1, D_IN), lambda i: (0, 0)),
        ],
        out_specs=pl.BlockSpec((blk, D_IN), lambda i: (i, 0)),
        out_shape=jax.ShapeDtypeStruct((n, D_IN), jnp.float32),
    )(aggs_lo, aggs_hi, xs_lo, xs_hi, degp,
      W_gcn, b_gcn.reshape(1, 512), W_res, b_res.reshape(1, 512),
      w_ih3, b_ih3, W_fc, b_fc.reshape(1, D_IN))
    return out
